# baseline (speedup 1.0000x reference)
import functools

import jax
import jax.numpy as jnp
from jax import lax
from jax.experimental import pallas as pl
from jax.experimental.pallas import tpu as pltpu
from jax.experimental.pallas import tpu_sc as plsc


def _sc_gather(table, idx):
    V, D = table.shape
    B = idx.shape[0]
    info = plsc.get_sparse_core_info()
    NC, NS = info.num_cores, info.num_subcores
    NW = NC * NS
    b_per_w = B // NW
    mesh = plsc.VectorSubcoreMesh(core_axis_name="c", subcore_axis_name="s")

    CH = 4
    c = b_per_w // CH

    @functools.partial(
        pl.kernel,
        mesh=mesh,
        out_type=jax.ShapeDtypeStruct((B, D), jnp.float32),
        scratch_types=[
            pltpu.VMEM((b_per_w,), jnp.int32),
            pltpu.VMEM((CH, c, D), jnp.float32),
            [pltpu.SemaphoreType.DMA] * CH,
            [pltpu.SemaphoreType.DMA] * CH,
        ],
    )
    def k(table_hbm, idx_hbm, out_hbm, idx_v, rows_v, gsems, wsems):
        wid = lax.axis_index("s") * NC + lax.axis_index("c")
        base = wid * b_per_w
        pltpu.sync_copy(idx_hbm.at[pl.ds(base, b_per_w)], idx_v)
        g = [
            pltpu.async_copy(
                table_hbm.at[idx_v.at[pl.ds(kk * c, c)]], rows_v.at[kk], gsems[kk]
            )
            for kk in range(CH)
        ]
        w = []
        for kk in range(CH):
            g[kk].wait()
            w.append(
                pltpu.async_copy(
                    rows_v.at[kk], out_hbm.at[pl.ds(base + kk * c, c)], wsems[kk]
                )
            )
        for kk in range(CH):
            w[kk].wait()

    return k(table, idx)


def _tc_matmul(x_u, coef_u, item, cvec, icpt, bm=8192):
    B, D = x_u.shape
    NI, DI = item.shape

    def body(x_ref, cu_ref, it_ref, cv_ref, ic_ref, o_ref):
        beta = jnp.concatenate(
            [jnp.zeros((1, D), jnp.float32), cu_ref[...]], axis=0
        )
        icpt_full = jnp.concatenate(
            [jnp.zeros((1, 1), jnp.float32), ic_ref[...]], axis=1
        )
        bias = (
            lax.dot_general(
                cv_ref[...], it_ref[...], (((1,), (1,)), ((), ())),
                preferred_element_type=jnp.float32,
            )
            + icpt_full
        )
        util = lax.dot_general(
            x_ref[...], beta, (((1,), (1,)), ((), ())),
            preferred_element_type=jnp.float32,
        )
        o_ref[...] = util + bias

    return pl.pallas_call(
        body,
        grid=(B // bm,),
        in_specs=[
            pl.BlockSpec((bm, D), lambda i: (i, 0)),
            pl.BlockSpec((NI - 1, D), lambda i: (0, 0)),
            pl.BlockSpec((NI, DI), lambda i: (0, 0)),
            pl.BlockSpec((1, DI), lambda i: (0, 0)),
            pl.BlockSpec((1, NI - 1), lambda i: (0, 0)),
        ],
        out_specs=pl.BlockSpec((bm, NI), lambda i: (i, 0)),
        out_shape=jax.ShapeDtypeStruct((B, NI), jnp.float32),
    )(x_u, coef_u, item, cvec, icpt)


def kernel(user_obs, item_obs, coef_user_obs, coef_item_obs, coef_intercept, user_index):
    NI, DI = item_obs.shape
    cvec = coef_item_obs.reshape(1, DI)
    icpt = coef_intercept.reshape(1, NI - 1)
    x_u = _sc_gather(user_obs, user_index)
    return _tc_matmul(x_u, coef_user_obs, item_obs, cvec, icpt)


# speedup vs baseline: 1.0123x; 1.0123x over previous
import functools

import jax
import jax.numpy as jnp
from jax import lax
from jax.experimental import pallas as pl
from jax.experimental.pallas import tpu as pltpu
from jax.experimental.pallas import tpu_sc as plsc


def _sc_gather(table, idx):
    V, D = table.shape
    B = idx.shape[0]
    info = plsc.get_sparse_core_info()
    NC, NS = info.num_cores, info.num_subcores
    NW = NC * NS
    b_per_w = B // NW
    mesh = plsc.VectorSubcoreMesh(core_axis_name="c", subcore_axis_name="s")

    @functools.partial(
        pl.kernel,
        mesh=mesh,
        out_type=jax.ShapeDtypeStruct((B, D), jnp.float32),
        scratch_types=[
            pltpu.VMEM((b_per_w,), jnp.int32),
            pltpu.VMEM((b_per_w, D), jnp.float32),
            pltpu.SemaphoreType.DMA,
            pltpu.SemaphoreType.DMA,
            pltpu.SemaphoreType.DMA,
        ],
    )
    def k(table_hbm, idx_hbm, out_hbm, idx_v, rows_v, gs0, gs1, ws):
        wid = lax.axis_index("s") * NC + lax.axis_index("c")
        base = wid * b_per_w
        h = b_per_w // 2
        pltpu.sync_copy(idx_hbm.at[pl.ds(base, b_per_w)], idx_v)
        g0 = pltpu.async_copy(
            table_hbm.at[idx_v.at[pl.ds(0, h)]], rows_v.at[pl.ds(0, h)], gs0
        )
        g1 = pltpu.async_copy(
            table_hbm.at[idx_v.at[pl.ds(h, h)]], rows_v.at[pl.ds(h, h)], gs1
        )
        g0.wait()
        w0 = pltpu.async_copy(
            rows_v.at[pl.ds(0, h)], out_hbm.at[pl.ds(base, h)], ws
        )
        g1.wait()
        w1 = pltpu.async_copy(
            rows_v.at[pl.ds(h, h)], out_hbm.at[pl.ds(base + h, h)], ws
        )
        w0.wait()
        w1.wait()

    return k(table, idx)


def _tc_matmul(x_u, coef_u, item, cvec, icpt, bm=8192):
    B, D = x_u.shape
    NI, DI = item.shape

    def body(x_ref, cu_ref, it_ref, cv_ref, ic_ref, o_ref):
        beta = jnp.concatenate(
            [jnp.zeros((1, D), jnp.float32), cu_ref[...]], axis=0
        )
        icpt_full = jnp.concatenate(
            [jnp.zeros((1, 1), jnp.float32), ic_ref[...]], axis=1
        )
        bias = (
            lax.dot_general(
                cv_ref[...], it_ref[...], (((1,), (1,)), ((), ())),
                preferred_element_type=jnp.float32,
            )
            + icpt_full
        )
        util = lax.dot_general(
            x_ref[...], beta, (((1,), (1,)), ((), ())),
            preferred_element_type=jnp.float32,
        )
        o_ref[...] = util + bias

    return pl.pallas_call(
        body,
        grid=(B // bm,),
        in_specs=[
            pl.BlockSpec((bm, D), lambda i: (i, 0)),
            pl.BlockSpec((NI - 1, D), lambda i: (0, 0)),
            pl.BlockSpec((NI, DI), lambda i: (0, 0)),
            pl.BlockSpec((1, DI), lambda i: (0, 0)),
            pl.BlockSpec((1, NI - 1), lambda i: (0, 0)),
        ],
        out_specs=pl.BlockSpec((bm, NI), lambda i: (i, 0)),
        out_shape=jax.ShapeDtypeStruct((B, NI), jnp.float32),
    )(x_u, coef_u, item, cvec, icpt)


def kernel(user_obs, item_obs, coef_user_obs, coef_item_obs, coef_intercept, user_index):
    NI, DI = item_obs.shape
    cvec = coef_item_obs.reshape(1, DI)
    icpt = coef_intercept.reshape(1, NI - 1)
    x_u = _sc_gather(user_obs, user_index)
    return _tc_matmul(x_u, coef_user_obs, item_obs, cvec, icpt)
